# baseline (speedup 1.0000x reference)
import functools

import numpy as np
import jax
import jax.numpy as jnp
from jax.experimental import pallas as pl

_INTERP = False

_C = 96
_H = 14
_W = 14
_L = _H * _W
_E = 3
_WIN = 2 * _E + 1
_K2 = _WIN * _WIN
_TOPK = 8
_PAD = 48
_LP = _L + 2 * _PAD
_B = 2
_BL = _B * _L

_OFFS = []
_VALID_NP = np.zeros((_L, 64), dtype=np.float32)
for _w in range(_K2):
    _dh = _w // _WIN - _E
    _dw = _w % _WIN - _E
    _OFFS.append(_dh * _W + _dw)
for _l in range(_L):
    _i, _j = _l // _W, _l % _W
    for _w in range(_K2):
        _dh = _w // _WIN - _E
        _dw = _w % _WIN - _E
        if 0 <= _i + _dh < _H and 0 <= _j + _dw < _W:
            _VALID_NP[_l, _w] = 1.0
_VALID = jnp.asarray(_VALID_NP)


def _k1_body(a_ref, w_ref, b_ref, g_ref, bt_ref, o_ref):
    y = jnp.dot(a_ref[...], w_ref[...], preferred_element_type=jnp.float32)
    y = y + b_ref[...]
    mean = jnp.mean(y, axis=0, keepdims=True)
    var = jnp.mean((y - mean) ** 2, axis=0, keepdims=True)
    y = (y - mean) * jax.lax.rsqrt(var + 1e-5) * g_ref[...] + bt_ref[...]
    o_ref[...] = jnp.maximum(y, 0.0)


def _k1b_body(brp_ref, valid_ref, o_ref):
    for b in range(_B):
        center = brp_ref[b, _PAD:_PAD + _L, :]
        cols = []
        for w in range(_K2):
            off = _OFFS[w]
            nb = brp_ref[b, _PAD + off:_PAD + off + _L, :]
            d2 = jnp.sum((center - nb) ** 2, axis=1, keepdims=True)
            cols.append(d2)
        d2m = jnp.concatenate(cols, axis=1)
        sim = 1.0 / (1.0 + jnp.sqrt(d2m))
        sim = jnp.concatenate(
            [sim, jnp.full((_L, 64 - _K2), 1e20, jnp.float32)], axis=1)
        o_ref[b] = jnp.where(valid_ref[...] > 0.0, sim, 1e20)


def _k2_body(s_ref, f_ref, o_ref):
    iota = jax.lax.broadcasted_iota(jnp.int32, (_L, 64), 1)
    for b in range(_B):
        vals = s_ref[b]
        sel = jnp.zeros((_L, 64), jnp.float32)
        for _ in range(_TOPK):
            mn = jnp.min(vals, axis=1, keepdims=True)
            eq = vals <= mn
            idx = jnp.min(jnp.where(eq, iota, 64), axis=1, keepdims=True)
            oh = iota == idx
            vals = jnp.where(oh, jnp.float32(3e38), vals)
            sel = sel + jnp.where(oh, 1.0, 0.0)
        acc = jnp.zeros((_L, _C), jnp.float32)
        for w in range(_K2):
            off = _OFFS[w]
            acc = acc + sel[:, w:w + 1] * f_ref[b, _PAD + off:_PAD + off + _L, :]
        o_ref[b] = acc / jnp.float32(_TOPK) - f_ref[b, _PAD:_PAD + _L, :]


def _k3_body(p_ref, w_ref, b_ref, g_ref, bt_ref, wa_ref, ga_ref, bta_ref,
             o_ref):
    y = jnp.dot(p_ref[...], w_ref[...], preferred_element_type=jnp.float32)
    y = y + b_ref[...]
    mean = jnp.mean(y, axis=0, keepdims=True)
    var = jnp.mean((y - mean) ** 2, axis=0, keepdims=True)
    y = (y - mean) * jax.lax.rsqrt(var + 1e-5) * g_ref[...] + bt_ref[...]
    feat = jnp.maximum(y, 0.0)
    s0 = jnp.mean(feat[0:_L], axis=0, keepdims=True)
    s1 = jnp.mean(feat[_L:_BL], axis=0, keepdims=True)
    att = jnp.concatenate([s0, s1], axis=0)
    att = jnp.dot(att, wa_ref[...], preferred_element_type=jnp.float32)
    m2 = jnp.mean(att, axis=0, keepdims=True)
    v2 = jnp.mean((att - m2) ** 2, axis=0, keepdims=True)
    att = (att - m2) * jax.lax.rsqrt(v2 + 1e-5) * ga_ref[...] + bta_ref[...]
    att = jax.nn.sigmoid(att)
    o_ref[...] = jnp.concatenate(
        [feat[0:_L] * att[0:1], feat[_L:_BL] * att[1:2]], axis=0)


def _k4_body(x_ref, rt_ref, o_ref):
    o_ref[...] = jnp.dot(x_ref[...], rt_ref[...],
                         preferred_element_type=jnp.float32)


def _call(body, out_shape, *args):
    return pl.pallas_call(
        body, out_shape=jax.ShapeDtypeStruct(out_shape, jnp.float32),
        interpret=_INTERP)(*args)


def kernel(x, Wd, bd, gd, betad, Wc, bc, gc, betac, Wa, ga, betaa):
    B, C, H, W, L = _B, _C, _H, _W, _L
    a = x.reshape(B, C, H, 4, W, 4).transpose(0, 2, 4, 1, 3, 5)
    a = a.reshape(B * L, C * 16)
    wd2 = Wd.reshape(C, C * 16).T
    xs_rows = _call(_k1_body, (_BL, C), a, wd2, bd.reshape(1, C),
                    gd.reshape(1, C), betad.reshape(1, C))
    xs = xs_rows.reshape(B, L, C)
    x1t = jnp.transpose(xs, (0, 2, 1))
    br = x1t.reshape(B, L, C)
    br_pad = jnp.pad(br, ((0, 0), (_PAD, _PAD), (0, 0)))
    sims = _call(_k1b_body, (B, L, 64), br_pad, _VALID)
    featp = jnp.pad(xs, ((0, 0), (_PAD, _PAD), (0, 0)))
    out_rows = _call(_k2_body, (B, L, C), sims, featp)
    x1_sp = x1t.reshape(B, C, H, W)
    out_sp = jnp.transpose(out_rows, (0, 2, 1)).reshape(B, C, H, W)
    cat = jnp.concatenate([x1_sp, out_sp], axis=1)
    catp = jnp.pad(cat, ((0, 0), (0, 0), (1, 1), (1, 1)))
    p = jnp.stack([catp[:, :, dy:dy + H, dx:dx + W]
                   for dy in range(3) for dx in range(3)], axis=2)
    p = p.transpose(0, 3, 4, 1, 2).reshape(_BL, 2 * C * 9)
    wc2 = Wc.reshape(C, 2 * C * 9).T
    wa2 = Wa.reshape(C, C).T
    feat_rows = _call(_k3_body, (_BL, C), p, wc2, bc.reshape(1, C),
                      gc.reshape(1, C), betac.reshape(1, C), wa2,
                      ga.reshape(1, C), betaa.reshape(1, C))
    X = feat_rows.reshape(B, L, C).transpose(0, 2, 1).reshape(B * C, L)
    mh = jax.image.resize(jnp.eye(H, dtype=jnp.float32), (H * 4, H),
                          method="bilinear")
    rt = jnp.kron(mh, mh).T
    y = _call(_k4_body, (B * C, L * 16), X, rt)
    return y.reshape(B, C, H * 4, W * 4)


# speedup vs baseline: 1.6161x; 1.6161x over previous
import functools

import numpy as np
import jax
import jax.numpy as jnp
from jax.experimental import pallas as pl

_INTERP = False

_C = 96
_H = 14
_W = 14
_L = _H * _W
_E = 3
_WIN = 2 * _E + 1
_K2 = _WIN * _WIN
_TOPK = 8
_PAD = 48
_B = 2
_BL = _B * _L

_OFFS = []
_VALID_NP = np.full((_L, 64), 0.0, dtype=np.float32)
for _w in range(_K2):
    _OFFS.append((_w // _WIN - _E) * _W + (_w % _WIN - _E))
for _l in range(_L):
    _i, _j = _l // _W, _l % _W
    for _w in range(_K2):
        _dh, _dw = _w // _WIN - _E, _w % _WIN - _E
        if 0 <= _i + _dh < _H and 0 <= _j + _dw < _W:
            _VALID_NP[_l, _w] = 1.0
_VALID = jnp.asarray(_VALID_NP)


def _ka_body(a_ref, w_ref, b_ref, g_ref, bt_ref, xs_ref, xt_ref):
    y = jnp.dot(a_ref[...], w_ref[...], preferred_element_type=jnp.float32)
    y = y + b_ref[...]
    mean = jnp.mean(y, axis=0, keepdims=True)
    var = jnp.mean((y - mean) ** 2, axis=0, keepdims=True)
    y = (y - mean) * jax.lax.rsqrt(var + 1e-5) * g_ref[...] + bt_ref[...]
    y = jnp.maximum(y, 0.0)
    xs_ref[...] = y
    for b in range(_B):
        xt_ref[b] = y[b * _L:(b + 1) * _L, :].T


def _ks_body(brp_ref, valid_ref, fp_ref, o_ref):
    iota = jax.lax.broadcasted_iota(jnp.int32, (_L, 64), 1)
    for b in range(_B):
        center = brp_ref[b, _PAD:_PAD + _L, :]
        cols = []
        for w in range(_K2):
            off = _OFFS[w]
            nb = brp_ref[b, _PAD + off:_PAD + off + _L, :]
            d2 = jnp.sum((center - nb) ** 2, axis=1, keepdims=True)
            cols.append(d2)
        d2m = jnp.concatenate(cols, axis=1)
        sim = 1.0 / (1.0 + jnp.sqrt(d2m))
        sim = jnp.concatenate(
            [sim, jnp.full((_L, 64 - _K2), 1e20, jnp.float32)], axis=1)
        vals = jnp.where(valid_ref[...] > 0.0, sim, 1e20)
        sel = jnp.zeros((_L, 64), jnp.float32)
        for _ in range(_TOPK):
            mn = jnp.min(vals, axis=1, keepdims=True)
            eq = vals <= mn
            idx = jnp.min(jnp.where(eq, iota, 64), axis=1, keepdims=True)
            oh = iota == idx
            vals = jnp.where(oh, jnp.float32(3e38), vals)
            sel = sel + jnp.where(oh, 1.0, 0.0)
        acc = jnp.zeros((_L, _C), jnp.float32)
        for w in range(_K2):
            off = _OFFS[w]
            acc = acc + sel[:, w:w + 1] * fp_ref[b, _PAD + off:_PAD + off + _L, :]
        o_ref[b] = acc / jnp.float32(_TOPK) - fp_ref[b, _PAD:_PAD + _L, :]


def _kb_body(xs_ref, or_ref, w_ref, b_ref, g_ref, bt_ref, wa_ref, ga_ref,
             bta_ref, rt_ref, o_ref):
    r = jax.lax.broadcasted_iota(jnp.int32, (_L, 1), 0)
    j = r % _W
    mL = (j >= 1).astype(jnp.float32)
    mR = (j <= _W - 2).astype(jnp.float32)
    zeros16 = jnp.zeros((16, 2 * _C), jnp.float32)
    feats = []
    for b in range(_B):
        cat = jnp.concatenate(
            [xs_ref[b * _L:(b + 1) * _L, :], or_ref[b]], axis=1)
        catp = jnp.concatenate([zeros16, cat, zeros16], axis=0)
        acc = jnp.zeros((_L, _C), jnp.float32)
        for dy in range(3):
            for dx in range(3):
                off = (dy - 1) * _W + (dx - 1)
                sh = catp[16 + off:16 + off + _L, :]
                ws = w_ref[(dy * 3 + dx) * 2 * _C:(dy * 3 + dx + 1) * 2 * _C, :]
                part = jnp.dot(sh, ws, preferred_element_type=jnp.float32)
                if dx == 0:
                    part = part * mL
                elif dx == 2:
                    part = part * mR
                acc = acc + part
        feats.append(acc + b_ref[...])
    mean = (jnp.sum(feats[0], axis=0, keepdims=True)
            + jnp.sum(feats[1], axis=0, keepdims=True)) / jnp.float32(_BL)
    var = (jnp.sum((feats[0] - mean) ** 2, axis=0, keepdims=True)
           + jnp.sum((feats[1] - mean) ** 2, axis=0, keepdims=True)) / jnp.float32(_BL)
    scale = jax.lax.rsqrt(var + 1e-5)
    f0 = jnp.maximum((feats[0] - mean) * scale * g_ref[...] + bt_ref[...], 0.0)
    f1 = jnp.maximum((feats[1] - mean) * scale * g_ref[...] + bt_ref[...], 0.0)
    att = jnp.concatenate([jnp.mean(f0, axis=0, keepdims=True),
                           jnp.mean(f1, axis=0, keepdims=True)], axis=0)
    att = jnp.dot(att, wa_ref[...], preferred_element_type=jnp.float32)
    m2 = jnp.mean(att, axis=0, keepdims=True)
    v2 = jnp.mean((att - m2) ** 2, axis=0, keepdims=True)
    att = (att - m2) * jax.lax.rsqrt(v2 + 1e-5) * ga_ref[...] + bta_ref[...]
    att = jax.nn.sigmoid(att)
    x = jnp.concatenate([(f0 * att[0:1]).T, (f1 * att[1:2]).T], axis=0)
    o_ref[...] = jnp.dot(x, rt_ref[...], preferred_element_type=jnp.float32)


def _call(body, out_shape, *args):
    if isinstance(out_shape, list):
        os = [jax.ShapeDtypeStruct(s, jnp.float32) for s in out_shape]
    else:
        os = jax.ShapeDtypeStruct(out_shape, jnp.float32)
    return pl.pallas_call(body, out_shape=os, interpret=_INTERP)(*args)


def kernel(x, Wd, bd, gd, betad, Wc, bc, gc, betac, Wa, ga, betaa):
    B, C, H, W, L = _B, _C, _H, _W, _L
    a = x.reshape(B, C, H, 4, W, 4).transpose(0, 2, 4, 1, 3, 5)
    a = a.reshape(B * L, C * 16)
    wd2 = Wd.reshape(C, C * 16).T
    xs_rows, x1t = _call(_ka_body, [(_BL, C), (B, C, L)], a, wd2,
                         bd.reshape(1, C), gd.reshape(1, C), betad.reshape(1, C))
    br = x1t.reshape(B, L, C)
    br_pad = jnp.pad(br, ((0, 0), (_PAD, _PAD), (0, 0)))
    featp = jnp.pad(xs_rows.reshape(B, L, C), ((0, 0), (_PAD, _PAD), (0, 0)))
    out_rows = _call(_ks_body, (B, L, C), br_pad, _VALID, featp)
    wc2 = Wc.transpose(2, 3, 1, 0).reshape(9 * 2 * C, C)
    wa2 = Wa.reshape(C, C).T
    mh = jax.image.resize(jnp.eye(H, dtype=jnp.float32), (H * 4, H),
                          method="bilinear")
    rt = jnp.kron(mh, mh).T
    y = _call(_kb_body, (B * C, L * 16), xs_rows, out_rows, wc2,
              bc.reshape(1, C), gc.reshape(1, C), betac.reshape(1, C), wa2,
              ga.reshape(1, C), betaa.reshape(1, C), rt)
    return y.reshape(B, C, H * 4, W * 4)
